# baseline (speedup 1.0000x reference)
import functools

import jax
import jax.numpy as jnp
from jax import lax
from jax.experimental import pallas as pl
from jax.experimental.pallas import tpu as pltpu


def _shift_lanes(x, delta):
    if delta == 0:
        return x
    k = delta % x.shape[-1]
    return jnp.concatenate([x[:, k:], x[:, :k]], axis=1)


def _fused_kernel(x_ref, w_ref, g_ref, b_ref, o_hbm, y_scr, st_scr,
                  ob_scr, sem, *, D, H, W, M, eps):
    HW = H * W
    S = D * HW
    p = pl.program_id(0)
    n = pl.program_id(1)
    N = pl.num_programs(1)

    @pl.when(p == 0)
    def conv_phase():
        xb = jnp.transpose(x_ref[...]).astype(jnp.bfloat16)
        cin = xb.shape[0]

        lane = lax.broadcasted_iota(jnp.int32, (1, S), 1)
        wp = lane % W
        hp = (lane // W) % H

        def _mask01(cond):
            return jnp.where(cond, jnp.float32(1), jnp.float32(0)).astype(
                jnp.bfloat16)

        mh = {0: _mask01(hp >= 2), 1: None, 2: _mask01(hp < H - 2)}
        mw = {0: _mask01(wp >= 2), 1: None, 2: _mask01(wp < W - 2)}

        taps = []
        for kh in range(3):
            for kw in range(3):
                dh, dw = 2 * kh - 2, 2 * kw - 2
                xs = _shift_lanes(xb, dh * W + dw)
                ms = [m for m in (mh[kh], mw[kw]) if m is not None]
                if len(ms) == 2:
                    xs = xs * (ms[0] * ms[1])
                elif ms:
                    xs = xs * ms[0]
                taps.append(xs)

        x9 = jnp.concatenate(taps, axis=0)
        zeros_d = jnp.zeros((9 * cin, HW), jnp.bfloat16)
        kt0 = jnp.concatenate([zeros_d, x9[:, : S - HW]], axis=1)
        kt2 = jnp.concatenate([x9[:, HW:], zeros_d], axis=1)
        rhs = jnp.concatenate([kt0, x9, kt2], axis=0)

        acc = jnp.dot(w_ref[...], rhs, preferred_element_type=jnp.float32)
        y_scr[n] = acc.astype(jnp.bfloat16)
        s = jnp.sum(acc, axis=1, keepdims=True)
        q = jnp.sum(acc * acc, axis=1, keepdims=True)
        st = jnp.concatenate([s, q], axis=1)

        @pl.when(n == 0)
        def _():
            st_scr[...] = st

        @pl.when(n > 0)
        def _():
            st_scr[...] = st_scr[...] + st

    @pl.when(p == 1)
    def bn_phase():
        st = st_scr[...]
        mean = st[:, 0:1] * (1.0 / M)
        var = jnp.maximum(st[:, 1:2] * (1.0 / M) - mean * mean, 0.0)
        sc = jnp.transpose(g_ref[...]) * lax.rsqrt(var + eps)
        sh = jnp.transpose(b_ref[...]) - mean * sc
        z = y_scr[n].astype(jnp.float32)
        o = jnp.maximum(z * sc + sh, 0.0)
        slot = lax.rem(n, 2)

        @pl.when(n >= 2)
        def _():
            pltpu.make_async_copy(
                ob_scr.at[slot], ob_scr.at[slot], sem.at[slot]).wait()

        ob_scr[slot] = jnp.transpose(o)
        cp = pltpu.make_async_copy(ob_scr.at[slot], o_hbm.at[n], sem.at[slot])
        cp.start()

        @pl.when(n == N - 1)
        def _():
            pltpu.make_async_copy(
                ob_scr.at[1 - slot], ob_scr.at[1 - slot],
                sem.at[1 - slot]).wait()
            pltpu.make_async_copy(
                ob_scr.at[slot], ob_scr.at[slot], sem.at[slot]).wait()


def kernel(x, weight, gamma, beta, eps=1e-5):
    N, Cin, D, H, W = x.shape
    Cout, _, KT, KH, KW = weight.shape
    T = KT * KH * KW
    S = D * H * W
    M = N * S

    xf = jnp.transpose(x, (0, 2, 3, 4, 1)).reshape(N, S, Cin)
    w_all = jnp.transpose(weight, (0, 2, 3, 4, 1)).reshape(
        Cout, T * Cin).astype(jnp.bfloat16)

    out = pl.pallas_call(
        functools.partial(_fused_kernel, D=D, H=H, W=W, M=M, eps=eps),
        grid=(2, N),
        in_specs=[
            pl.BlockSpec((None, S, Cin),
                         lambda p, n: ((1 - p) * n + p * (N - 1), 0, 0)),
            pl.BlockSpec((Cout, T * Cin), lambda p, n: (0, 0)),
            pl.BlockSpec((1, Cout), lambda p, n: (0, 0)),
            pl.BlockSpec((1, Cout), lambda p, n: (0, 0)),
        ],
        out_specs=pl.BlockSpec(memory_space=pl.ANY),
        out_shape=jax.ShapeDtypeStruct((N, S, Cout), jnp.float32),
        scratch_shapes=[
            pltpu.VMEM((N, Cout, S), jnp.bfloat16),
            pltpu.VMEM((Cout, 2), jnp.float32),
            pltpu.VMEM((2, S, Cout), jnp.float32),
            pltpu.SemaphoreType.DMA((2,)),
        ],
        compiler_params=pltpu.CompilerParams(
            dimension_semantics=("arbitrary", "arbitrary"),
            vmem_limit_bytes=63 * 1024 * 1024),
    )(xf, w_all, gamma.astype(jnp.float32)[None, :],
      beta.astype(jnp.float32)[None, :])

    return jnp.transpose(out.reshape(N, D, H, W, Cout), (0, 4, 1, 2, 3))


# speedup vs baseline: 1.0760x; 1.0760x over previous
import functools

import jax
import jax.numpy as jnp
from jax import lax
from jax.experimental import pallas as pl
from jax.experimental.pallas import tpu as pltpu


def _shift_lanes(x, delta):
    if delta == 0:
        return x
    k = delta % x.shape[-1]
    return jnp.concatenate([x[:, k:], x[:, :k]], axis=1)


def _fused_kernel(x_ref, w_ref, g_ref, b_ref, o_ref, y_scr, st_scr,
                  *, D, H, W, M, eps):
    HW = H * W
    S = D * HW
    p = pl.program_id(0)
    n = pl.program_id(1)

    @pl.when(p == 0)
    def conv_phase():
        xb = jnp.transpose(x_ref[...].astype(jnp.bfloat16))
        cin = xb.shape[0]

        lane = lax.broadcasted_iota(jnp.int32, (1, S), 1)
        wp = lane % W
        hp = (lane // W) % H

        def _mask01(cond):
            return jnp.where(cond, jnp.float32(1), jnp.float32(0)).astype(
                jnp.bfloat16)

        mh = {0: _mask01(hp >= 2), 1: None, 2: _mask01(hp < H - 2)}
        mw = {0: _mask01(wp >= 2), 1: None, 2: _mask01(wp < W - 2)}

        taps = []
        for kh in range(3):
            for kw in range(3):
                dh, dw = 2 * kh - 2, 2 * kw - 2
                xs = _shift_lanes(xb, dh * W + dw)
                ms = [m for m in (mh[kh], mw[kw]) if m is not None]
                if len(ms) == 2:
                    xs = xs * (ms[0] * ms[1])
                elif ms:
                    xs = xs * ms[0]
                taps.append(xs)

        x9 = jnp.concatenate(taps, axis=0)
        zeros_d = jnp.zeros((9 * cin, HW), jnp.bfloat16)
        kt0 = jnp.concatenate([zeros_d, x9[:, : S - HW]], axis=1)
        kt2 = jnp.concatenate([x9[:, HW:], zeros_d], axis=1)
        rhs = jnp.concatenate([kt0, x9, kt2], axis=0)

        acc = jnp.dot(w_ref[...], rhs, preferred_element_type=jnp.float32)
        y_scr[n] = acc.astype(jnp.bfloat16)
        s = jnp.sum(acc, axis=1, keepdims=True)
        q = jnp.sum(acc * acc, axis=1, keepdims=True)
        st = jnp.concatenate([s, q], axis=1)

        @pl.when(n == 0)
        def _():
            st_scr[...] = st

        @pl.when(n > 0)
        def _():
            st_scr[...] = st_scr[...] + st

    @pl.when(p == 1)
    def bn_phase():
        st_rows = jnp.transpose(st_scr[...])
        mean = st_rows[0:1] * (1.0 / M)
        var = jnp.maximum(st_rows[1:2] * (1.0 / M) - mean * mean, 0.0)
        sc = g_ref[...] * lax.rsqrt(var + eps)
        sh = b_ref[...] - mean * sc
        zt = jnp.transpose(y_scr[n])
        o_ref[...] = jnp.maximum(zt.astype(jnp.float32) * sc + sh, 0.0)


def kernel(x, weight, gamma, beta, eps=1e-5):
    N, Cin, D, H, W = x.shape
    Cout, _, KT, KH, KW = weight.shape
    T = KT * KH * KW
    S = D * H * W
    M = N * S

    xf = jnp.transpose(x, (0, 2, 3, 4, 1)).reshape(N, S, Cin)
    w_all = jnp.transpose(weight, (0, 2, 3, 4, 1)).reshape(
        Cout, T * Cin).astype(jnp.bfloat16)

    out = pl.pallas_call(
        functools.partial(_fused_kernel, D=D, H=H, W=W, M=M, eps=eps),
        grid=(2, N),
        in_specs=[
            pl.BlockSpec((None, S, Cin),
                         lambda p, n: ((1 - p) * n + p * (N - 1), 0, 0)),
            pl.BlockSpec((Cout, T * Cin), lambda p, n: (0, 0)),
            pl.BlockSpec((1, Cout), lambda p, n: (0, 0)),
            pl.BlockSpec((1, Cout), lambda p, n: (0, 0)),
        ],
        out_specs=pl.BlockSpec((None, S, Cout), lambda p, n: (p * n, 0, 0)),
        out_shape=jax.ShapeDtypeStruct((N, S, Cout), jnp.float32),
        scratch_shapes=[
            pltpu.VMEM((N, Cout, S), jnp.bfloat16),
            pltpu.VMEM((Cout, 2), jnp.float32),
        ],
        compiler_params=pltpu.CompilerParams(
            dimension_semantics=("arbitrary", "arbitrary"),
            vmem_limit_bytes=63 * 1024 * 1024),
    )(xf, w_all, gamma.astype(jnp.float32)[None, :],
      beta.astype(jnp.float32)[None, :])

    return jnp.transpose(out.reshape(N, D, H, W, Cout), (0, 4, 1, 2, 3))
